# baseline (speedup 1.0000x reference)
import functools

import jax
import jax.numpy as jnp
from jax.experimental import pallas as pl
from jax.experimental.pallas import tpu as pltpu

N = 512
B = 8
T_IN = 12
T_OUT = 12
F_IN = 2
H = 32
TG = 4
NG = B * (T_IN // TG)
BN_EPS = 1e-5

_HIGHEST = jax.lax.Precision.HIGHEST


def _dot_a3(x, rhs_h, rhs_l):
    f32 = jnp.float32
    x_h = x.astype(jnp.bfloat16)
    x_l = (x - x_h.astype(f32)).astype(jnp.bfloat16)
    return (jnp.dot(x_h, rhs_h, preferred_element_type=f32)
            + (jnp.dot(x_h, rhs_l, preferred_element_type=f32)
               + jnp.dot(x_l, rhs_h, preferred_element_type=f32)))


def _tgcn_kernel(xp_ref, xg_ref, at_ref, gamma_ref, beta_ref,
                 w1t_ref, b1t_ref, w2dt_ref, b2t_ref,
                 wih_ref, whh_ref, bih_ref, bhh_ref,
                 wlin_ref, blin_ref,
                 out_ref, gi_buf):
    f32 = jnp.float32

    xp = xp_ref[...]
    m = jnp.mean(xp, axis=0, keepdims=True)
    xc = xp - m
    v = jnp.mean(xc * xc, axis=0, keepdims=True)
    s = gamma_ref[...] * jax.lax.rsqrt(v + BN_EPS)
    c = beta_ref[...] - s * m

    at = at_ref[...]
    at_h = at.astype(jnp.bfloat16)
    at_l = (at - at_h.astype(f32)).astype(jnp.bfloat16)
    w1t = w1t_ref[...]
    w2dt = w2dt_ref[...]
    b1t = b1t_ref[...]
    b2t = b2t_ref[...]
    wih = wih_ref[...]
    whh = whh_ref[...]
    bih = bih_ref[...]
    bhh = bhh_ref[...]

    for g in range(NG):
        b, j = g // 3, g % 3
        xg = xg_ref[g]
        bn = xg * s + c
        blocks = []
        for i in range(TG):
            blk = (w1t[:, 0:1] * bn[i:i + 1, :]
                   + w1t[:, 1:2] * bn[TG + i:TG + i + 1, :])
            blocks.append(blk)
        y1t = jnp.concatenate(blocks, axis=0)
        t2t = jnp.dot(y1t, at, preferred_element_type=f32,
                      precision=_HIGHEST) + b1t
        t3t = jnp.maximum(t2t, 0.0)
        zt = jnp.dot(w2dt, t3t, preferred_element_type=f32,
                     precision=_HIGHEST)
        t4t = jnp.dot(zt, at, preferred_element_type=f32,
                      precision=_HIGHEST) + b2t
        st = jax.nn.sigmoid(t4t)
        for i in range(TG):
            gi = jnp.dot(wih, st[H * i:H * (i + 1), :],
                         preferred_element_type=f32)
            gi_buf[TG * j + i, :, N * b:N * (b + 1)] = gi + bih

    wlin = wlin_ref[...]
    blin = blin_ref[...]
    NC = 2
    CW = B * N // NC
    hs = [jnp.zeros((H, CW), dtype=f32) for _ in range(NC)]
    for k in range(T_OUT + 1):
        for t in range(T_IN):
            for ci in range(NC):
                h = hs[ci]
                lo, hi = ci * CW, (ci + 1) * CW
                gh = jnp.dot(whh, h, preferred_element_type=f32) + bhh
                g = gi_buf[t, :, lo:hi]
                rz = jax.nn.sigmoid(g[0:2 * H] + gh[0:2 * H])
                r = rz[0:H]
                z = rz[H:2 * H]
                n = jnp.tanh(g[2 * H:3 * H] + r * gh[2 * H:3 * H])
                h = n + z * (h - n)
                hs[ci] = h
                if k < T_OUT:
                    gi_buf[t, :, lo:hi] = jnp.dot(
                        wih, h, preferred_element_type=f32) + bih
                if k >= 1 and t == 0:
                    out_ref[k - 1:k, lo:hi] = (jnp.sum(h * wlin, axis=0,
                                                       keepdims=True) + blin)


@functools.partial(jax.jit, static_argnames=())
def kernel(A, X, bn_gamma, bn_beta, W1, b1, W2, b2,
           W_ih, W_hh, b_ih, b_hh, W_lin, b_lin):
    f32 = jnp.float32
    xpt = jnp.transpose(X, (0, 2, 3, 1)).reshape(B * T_IN * F_IN, N)
    xg = (jnp.transpose(X, (0, 2, 3, 1))
          .reshape(B, T_IN // TG, TG, F_IN, N)
          .transpose(0, 1, 3, 2, 4)
          .reshape(NG, F_IN * TG, N))
    at = A.T
    gamma2 = bn_gamma.reshape(1, N)
    beta2 = bn_beta.reshape(1, N)
    w1t = W1.T
    b1t = jnp.tile(b1, TG).reshape(TG * H, 1)
    w2dt = jnp.kron(jnp.eye(TG, dtype=f32), W2.T)
    b2t = jnp.tile(b2, TG).reshape(TG * H, 1)
    bih = b_ih.reshape(3 * H, 1)
    bhh = b_hh.reshape(3 * H, 1)
    wlin = W_lin.reshape(H, 1)
    blin = b_lin.reshape(1, 1)

    out = pl.pallas_call(
        _tgcn_kernel,
        out_shape=jax.ShapeDtypeStruct((T_OUT, B * N), f32),
        scratch_shapes=[pltpu.VMEM((T_IN, 3 * H, B * N), f32)],
    )(xpt, xg, at, gamma2, beta2, w1t, b1t, w2dt, b2t,
      W_ih, W_hh, bih, bhh, wlin, blin)

    return jnp.transpose(out).reshape(B, N, T_OUT)


# speedup vs baseline: 1.2929x; 1.2929x over previous
import functools

import jax
import jax.numpy as jnp
from jax.experimental import pallas as pl
from jax.experimental.pallas import tpu as pltpu

N = 512
B = 8
T_IN = 12
T_OUT = 12
F_IN = 2
H = 32
TG = 4
NG = B * (T_IN // TG)
BN_EPS = 1e-5

_HIGHEST = jax.lax.Precision.HIGHEST


def _tgcn_kernel(xp_ref, xg_ref, at_ref, gamma_ref, beta_ref,
                 w1t_ref, b1t_ref, w2dt_ref, b2t_ref,
                 wrz_ref, win_ref, whn_ref, brz_ref, bin_ref, bhn_ref,
                 wlin_ref, blin_ref,
                 out_ref, buf):
    f32 = jnp.float32

    xp = xp_ref[...]
    m = jnp.mean(xp, axis=0, keepdims=True)
    xc = xp - m
    v = jnp.mean(xc * xc, axis=0, keepdims=True)
    s = gamma_ref[...] * jax.lax.rsqrt(v + BN_EPS)
    c = beta_ref[...] - s * m

    at = at_ref[...]
    w1t = w1t_ref[...]
    w2dt = w2dt_ref[...]
    b1t = b1t_ref[...]
    b2t = b2t_ref[...]

    for g in range(NG):
        b, j = g // 3, g % 3
        xg = xg_ref[g]
        bn = xg * s + c
        blocks = []
        for i in range(TG):
            blk = (w1t[:, 0:1] * bn[i:i + 1, :]
                   + w1t[:, 1:2] * bn[TG + i:TG + i + 1, :])
            blocks.append(blk)
        y1t = jnp.concatenate(blocks, axis=0)
        t2t = jnp.dot(y1t, at, preferred_element_type=f32,
                      precision=_HIGHEST) + b1t
        t3t = jnp.maximum(t2t, 0.0)
        zt = jnp.dot(w2dt, t3t, preferred_element_type=f32,
                     precision=_HIGHEST)
        t4t = jnp.dot(zt, at, preferred_element_type=f32,
                      precision=_HIGHEST) + b2t
        st = jax.nn.sigmoid(t4t)
        for i in range(TG):
            buf[TG * j + i, 0:H, N * b:N * (b + 1)] = st[H * i:H * (i + 1), :]

    wrz = wrz_ref[...]
    win = win_ref[...]
    whn = whn_ref[...]
    brz = brz_ref[...]
    bin_ = bin_ref[...]
    bhn = bhn_ref[...]
    wlin = wlin_ref[...]
    blin = blin_ref[...]

    h = jnp.zeros((H, B * N), dtype=f32)
    for k in range(T_OUT + 1):
        for t in range(T_IN):
            buf[t, H:2 * H, :] = h
            xh = buf[t]
            rz = jax.nn.sigmoid(
                jnp.dot(wrz, xh, preferred_element_type=f32) + brz)
            i_n = jnp.dot(win, xh[0:H], preferred_element_type=f32)
            h_n = jnp.dot(whn, xh[H:2 * H], preferred_element_type=f32)
            n = jnp.tanh((i_n + bin_) + rz[0:H] * (h_n + bhn))
            h = n + rz[H:2 * H] * (h - n)
            if k < T_OUT:
                buf[t, 0:H, :] = h
            if k >= 1 and t == 0:
                out_ref[k - 1:k, :] = (jnp.sum(h * wlin, axis=0,
                                               keepdims=True) + blin)


@functools.partial(jax.jit, static_argnames=())
def kernel(A, X, bn_gamma, bn_beta, W1, b1, W2, b2,
           W_ih, W_hh, b_ih, b_hh, W_lin, b_lin):
    f32 = jnp.float32
    xpt = jnp.transpose(X, (0, 2, 3, 1)).reshape(B * T_IN * F_IN, N)
    xg = (jnp.transpose(X, (0, 2, 3, 1))
          .reshape(B, T_IN // TG, TG, F_IN, N)
          .transpose(0, 1, 3, 2, 4)
          .reshape(NG, F_IN * TG, N))
    at = A.T
    gamma2 = bn_gamma.reshape(1, N)
    beta2 = bn_beta.reshape(1, N)
    w1t = W1.T
    b1t = jnp.tile(b1, TG).reshape(TG * H, 1)
    w2dt = jnp.kron(jnp.eye(TG, dtype=f32), W2.T)
    b2t = jnp.tile(b2, TG).reshape(TG * H, 1)
    wrz = jnp.concatenate([W_ih[0:2 * H], W_hh[0:2 * H]], axis=1)
    win = W_ih[2 * H:3 * H]
    whn = W_hh[2 * H:3 * H]
    brz = (b_ih[0:2 * H] + b_hh[0:2 * H]).reshape(2 * H, 1)
    bin_ = b_ih[2 * H:3 * H].reshape(H, 1)
    bhn = b_hh[2 * H:3 * H].reshape(H, 1)
    wlin = W_lin.reshape(H, 1)
    blin = b_lin.reshape(1, 1)

    out = pl.pallas_call(
        _tgcn_kernel,
        out_shape=jax.ShapeDtypeStruct((T_OUT, B * N), f32),
        scratch_shapes=[pltpu.VMEM((T_IN, 2 * H, B * N), f32)],
    )(xpt, xg, at, gamma2, beta2, w1t, b1t, w2dt, b2t,
      wrz, win, whn, brz, bin_, bhn, wlin, blin)

    return jnp.transpose(out).reshape(B, N, T_OUT)


# speedup vs baseline: 1.3383x; 1.0351x over previous
import functools

import jax
import jax.numpy as jnp
from jax.experimental import pallas as pl
from jax.experimental.pallas import tpu as pltpu

N = 512
B = 8
T_IN = 12
T_OUT = 12
F_IN = 2
H = 32
TG = 4
NG = B * (T_IN // TG)
BN_EPS = 1e-5

_HIGHEST = jax.lax.Precision.HIGHEST


def _tgcn_kernel(xp_ref, xg_ref, at_ref, gamma_ref, beta_ref,
                 w1t_ref, b1t_ref, w2dt_ref, b2t_ref,
                 wrz_ref, win_ref, whn_ref, brz_ref, bin_ref, bhn_ref,
                 wlin_ref, blin_ref,
                 out_ref, buf):
    f32 = jnp.float32

    xp = xp_ref[...]
    m = jnp.mean(xp, axis=0, keepdims=True)
    xc = xp - m
    v = jnp.mean(xc * xc, axis=0, keepdims=True)
    s = gamma_ref[...] * jax.lax.rsqrt(v + BN_EPS)
    c = beta_ref[...] - s * m

    at = at_ref[...]
    w1t = w1t_ref[...]
    w2dt = w2dt_ref[...]
    b1t = b1t_ref[...]
    b2t = b2t_ref[...]

    for g in range(NG):
        b, j = g // 3, g % 3
        xg = xg_ref[g]
        bn = xg * s + c
        blocks = []
        for i in range(TG):
            blk = (w1t[:, 0:1] * bn[i:i + 1, :]
                   + w1t[:, 1:2] * bn[TG + i:TG + i + 1, :])
            blocks.append(blk)
        y1t = jnp.concatenate(blocks, axis=0)
        t2t = jnp.dot(y1t, at, preferred_element_type=f32,
                      precision=_HIGHEST) + b1t
        t3t = jnp.maximum(t2t, 0.0)
        zt = jnp.dot(w2dt, t3t, preferred_element_type=f32,
                     precision=_HIGHEST)
        t4t = jnp.dot(zt, at, preferred_element_type=f32,
                      precision=_HIGHEST) + b2t
        st = jax.nn.sigmoid(t4t)
        for i in range(TG):
            buf[TG * j + i, 0:H, N * b:N * (b + 1)] = st[H * i:H * (i + 1), :]

    wrz = wrz_ref[...]
    win = win_ref[...]
    whn = whn_ref[...]
    brz = brz_ref[...]
    bin_ = bin_ref[...]
    bhn = bhn_ref[...]
    wlin = wlin_ref[...]
    blin = blin_ref[...]

    buf[0, H:2 * H, :] = jnp.zeros((H, B * N), dtype=f32)
    for k in range(T_OUT + 1):
        n_steps = T_IN if k < T_OUT else 1
        for t in range(n_steps):
            xh = buf[t]
            rz = jax.nn.sigmoid(
                jnp.dot(wrz, xh, preferred_element_type=f32) + brz)
            i_n = jnp.dot(win, xh[0:H], preferred_element_type=f32)
            h_n = jnp.dot(whn, xh[H:2 * H], preferred_element_type=f32)
            n = jnp.tanh((i_n + bin_) + rz[0:H] * (h_n + bhn))
            h = n + rz[H:2 * H] * (xh[H:2 * H] - n)
            last_step = (k == T_OUT)
            if not last_step:
                nt = t + 1 if t + 1 < T_IN else 0
                buf[nt, H:2 * H, :] = h
            if k < T_OUT - 1 or (k == T_OUT - 1 and t == 0):
                buf[t, 0:H, :] = h
            if k >= 1 and t == 0:
                out_ref[k - 1:k, :] = (jnp.sum(h * wlin, axis=0,
                                               keepdims=True) + blin)


@functools.partial(jax.jit, static_argnames=())
def kernel(A, X, bn_gamma, bn_beta, W1, b1, W2, b2,
           W_ih, W_hh, b_ih, b_hh, W_lin, b_lin):
    f32 = jnp.float32
    xpt = jnp.transpose(X, (0, 2, 3, 1)).reshape(B * T_IN * F_IN, N)
    xg = (jnp.transpose(X, (0, 2, 3, 1))
          .reshape(B, T_IN // TG, TG, F_IN, N)
          .transpose(0, 1, 3, 2, 4)
          .reshape(NG, F_IN * TG, N))
    at = A.T
    gamma2 = bn_gamma.reshape(1, N)
    beta2 = bn_beta.reshape(1, N)
    w1t = W1.T
    b1t = jnp.tile(b1, TG).reshape(TG * H, 1)
    w2dt = jnp.kron(jnp.eye(TG, dtype=f32), W2.T)
    b2t = jnp.tile(b2, TG).reshape(TG * H, 1)
    wrz = jnp.concatenate([W_ih[0:2 * H], W_hh[0:2 * H]], axis=1)
    win = W_ih[2 * H:3 * H]
    whn = W_hh[2 * H:3 * H]
    brz = (b_ih[0:2 * H] + b_hh[0:2 * H]).reshape(2 * H, 1)
    bin_ = b_ih[2 * H:3 * H].reshape(H, 1)
    bhn = b_hh[2 * H:3 * H].reshape(H, 1)
    wlin = W_lin.reshape(H, 1)
    blin = b_lin.reshape(1, 1)

    out = pl.pallas_call(
        _tgcn_kernel,
        out_shape=jax.ShapeDtypeStruct((T_OUT, B * N), f32),
        scratch_shapes=[pltpu.VMEM((T_IN, 2 * H, B * N), f32)],
    )(xpt, xg, at, gamma2, beta2, w1t, b1t, w2dt, b2t,
      wrz, win, whn, brz, bin_, bhn, wlin, blin)

    return jnp.transpose(out).reshape(B, N, T_OUT)


# speedup vs baseline: 1.5989x; 1.1947x over previous
import functools

import jax
import jax.numpy as jnp
from jax.experimental import pallas as pl
from jax.experimental.pallas import tpu as pltpu

N = 512
B = 8
T_IN = 12
T_OUT = 12
F_IN = 2
H = 32
TG = 4
NG = B * (T_IN // TG)
BN_EPS = 1e-5

_HIGHEST = jax.lax.Precision.HIGHEST


def _tgcn_kernel(xp_ref, xg_ref, at_ref, gamma_ref, beta_ref,
                 w1t_ref, b1t_ref, w2dt_ref, b2t_ref,
                 wall_ref, wlin_ref, blin_ref,
                 out_ref, buf):
    f32 = jnp.float32

    xp = xp_ref[...]
    m = jnp.mean(xp, axis=0, keepdims=True)
    xc = xp - m
    v = jnp.mean(xc * xc, axis=0, keepdims=True)
    s = gamma_ref[...] * jax.lax.rsqrt(v + BN_EPS)
    c = beta_ref[...] - s * m

    at = at_ref[...]
    w1t = w1t_ref[...]
    w2dt = w2dt_ref[...]
    b1t = b1t_ref[...]
    b2t = b2t_ref[...]

    for g in range(NG):
        b, j = g // 3, g % 3
        xg = xg_ref[g]
        bn = xg * s + c
        blocks = []
        for i in range(TG):
            blk = (w1t[:, 0:1] * bn[i:i + 1, :]
                   + w1t[:, 1:2] * bn[TG + i:TG + i + 1, :])
            blocks.append(blk)
        y1t = jnp.concatenate(blocks, axis=0)
        t2t = jnp.dot(y1t, at, preferred_element_type=f32,
                      precision=_HIGHEST) + b1t
        t3t = jnp.maximum(t2t, 0.0)
        zt = jnp.dot(w2dt, t3t, preferred_element_type=f32,
                     precision=_HIGHEST)
        t4t = jnp.dot(zt, at, preferred_element_type=f32,
                      precision=_HIGHEST) + b2t
        st = jax.nn.sigmoid(t4t)
        for i in range(TG):
            buf[TG * j + i, 0:H, N * b:N * (b + 1)] = st[H * i:H * (i + 1), :]

    wall = wall_ref[...]
    wlin = wlin_ref[...]
    blin = blin_ref[...]

    buf[0, H:2 * H, :] = jnp.zeros((H, B * N), dtype=f32)
    buf[:, 2 * H:2 * H + 2, :] = jnp.ones((T_IN, 2, B * N), dtype=f32)
    for k in range(T_OUT + 1):
        n_steps = T_IN if k < T_OUT else 1
        for t in range(n_steps):
            xh = buf[t]
            g = jnp.dot(wall, xh, preferred_element_type=f32)
            rz = jax.nn.sigmoid(g[0:2 * H])
            n = jnp.tanh(g[2 * H:3 * H] + rz[0:H] * g[3 * H:4 * H])
            h = n + rz[H:2 * H] * (xh[H:2 * H] - n)
            last_step = (k == T_OUT)
            if not last_step:
                nt = t + 1 if t + 1 < T_IN else 0
                buf[nt, H:2 * H, :] = h
            if k < T_OUT - 1 or (k == T_OUT - 1 and t == 0):
                buf[t, 0:H, :] = h
            if k >= 1 and t == 0:
                out_ref[k - 1:k, :] = (jnp.sum(h * wlin, axis=0,
                                               keepdims=True) + blin)


@functools.partial(jax.jit, static_argnames=())
def kernel(A, X, bn_gamma, bn_beta, W1, b1, W2, b2,
           W_ih, W_hh, b_ih, b_hh, W_lin, b_lin):
    f32 = jnp.float32
    xpt = jnp.transpose(X, (0, 2, 3, 1)).reshape(B * T_IN * F_IN, N)
    xg = (jnp.transpose(X, (0, 2, 3, 1))
          .reshape(B, T_IN // TG, TG, F_IN, N)
          .transpose(0, 1, 3, 2, 4)
          .reshape(NG, F_IN * TG, N))
    at = A.T
    gamma2 = bn_gamma.reshape(1, N)
    beta2 = bn_beta.reshape(1, N)
    w1t = W1.T
    b1t = jnp.tile(b1, TG).reshape(TG * H, 1)
    w2dt = jnp.kron(jnp.eye(TG, dtype=f32), W2.T)
    b2t = jnp.tile(b2, TG).reshape(TG * H, 1)
    zeros_h = jnp.zeros((H, H), dtype=f32)
    ball = jnp.concatenate([
        (b_ih[0:2 * H] + b_hh[0:2 * H]),
        b_ih[2 * H:3 * H],
        b_hh[2 * H:3 * H],
    ]).reshape(4 * H, 1)
    bhi = ball.astype(jnp.bfloat16).astype(f32)
    blo = ball - bhi
    wall = jnp.concatenate([
        jnp.concatenate([W_ih[0:2 * H], W_hh[0:2 * H]], axis=1),
        jnp.concatenate([W_ih[2 * H:3 * H], zeros_h], axis=1),
        jnp.concatenate([zeros_h, W_hh[2 * H:3 * H]], axis=1),
    ], axis=0)
    wall = jnp.concatenate([wall, bhi, blo], axis=1)
    wlin = W_lin.reshape(H, 1)
    blin = b_lin.reshape(1, 1)

    out = pl.pallas_call(
        _tgcn_kernel,
        out_shape=jax.ShapeDtypeStruct((T_OUT, B * N), f32),
        scratch_shapes=[pltpu.VMEM((T_IN, 2 * H + 2, B * N), f32)],
    )(xpt, xg, at, gamma2, beta2, w1t, b1t, w2dt, b2t,
      wall, wlin, blin)

    return jnp.transpose(out).reshape(B, N, T_OUT)
